# baseline (speedup 1.0000x reference)
import functools

import jax
import jax.numpy as jnp
from jax import lax
from jax.experimental import pallas as pl
from jax.experimental.pallas import tpu as pltpu
from jax.experimental.pallas import tpu_sc as plsc

EMBED = 64
BATCH = 16384
NENT = 1000000
NREL = 1000
NC = 2
NS = 16
D_PER_C = EMBED // NC
B_PER_T = BATCH // NS
SUB = 128
NSUB = B_PER_T // SUB
LANES = 16
NREL_P = 1024

ALIGNED = (NENT // SUB) * SUB
TAIL = SUB
ROW_P = ALIGNED + TAIL
_SLICE_SIZES = [977 * SUB] * 4 + [976 * SUB] * 4
_SLICE_OFFS = [sum(_SLICE_SIZES[:i]) for i in range(8)]

_mesh = plsc.VectorSubcoreMesh(
    core_axis_name="c", subcore_axis_name="s", num_cores=NC, num_subcores=NS
)


@functools.partial(
    pl.kernel,
    mesh=_mesh,
    out_type=jax.ShapeDtypeStruct((NC * BATCH,), jnp.float32),
    scratch_types=[
        pltpu.VMEM((NSUB, SUB), jnp.int32),
        pltpu.VMEM((NSUB, SUB), jnp.int32),
        pltpu.VMEM((NSUB, SUB), jnp.int32),
        pltpu.VMEM((NSUB, SUB), jnp.int32),
        pltpu.VMEM((NSUB, SUB), jnp.float32),
        pltpu.VMEM((NSUB, SUB), jnp.float32),
        pltpu.VMEM((NSUB, SUB), jnp.float32),
        pltpu.VMEM((NSUB, SUB), jnp.float32),
        pltpu.VMEM_SHARED((ROW_P,), jnp.float32),
        pltpu.VMEM_SHARED((EMBED * NREL_P,), jnp.float32),
        pltpu.SemaphoreType.DMA,
        pltpu.SemaphoreType.DMA,
    ],
)
def _transe_sc(h_hbm, r_hbm, t_hbm, ent_t_hbm, rel_t_hbm, tail_t_hbm,
               out_hbm, hi, ri, ti, rsi, hv, tv, relv, acc, srow,
               srel, sem, sem_s):
    c = lax.axis_index("c")
    s = lax.axis_index("s")
    tbase = s * B_PER_T
    dbase = c * D_PER_C

    def fire_stream(row):
        for st in range(8):
            @pl.when(s == st)
            def _():
                sl = pl.ds(_SLICE_OFFS[st], _SLICE_SIZES[st])
                pltpu.async_copy(ent_t_hbm.at[row].at[sl],
                                 srow.at[sl], sem_s)

        @pl.when(s == 8)
        def _():
            pltpu.async_copy(tail_t_hbm.at[row],
                             srow.at[pl.ds(ALIGNED, TAIL)], sem_s)

    def wait_stream():
        for st in range(8):
            @pl.when(s == st)
            def _():
                sl = pl.ds(_SLICE_OFFS[st], _SLICE_SIZES[st])
                pltpu.make_async_copy(ent_t_hbm.at[0].at[sl],
                                      srow.at[sl], sem_s).wait()

        @pl.when(s == 8)
        def _():
            pltpu.make_async_copy(
                tail_t_hbm.at[0],
                srow.at[pl.ds(ALIGNED, TAIL)], sem_s
            ).wait()

    idx_copies = []
    for k in range(NSUB):
        off = tbase + k * SUB
        idx_copies.append(
            pltpu.async_copy(h_hbm.at[pl.ds(off, SUB)], hi.at[k], sem))
        idx_copies.append(
            pltpu.async_copy(r_hbm.at[pl.ds(off, SUB)], ri.at[k], sem))
        idx_copies.append(
            pltpu.async_copy(t_hbm.at[pl.ds(off, SUB)], ti.at[k], sem))
    for cp in idx_copies:
        cp.wait()

    fire_stream(dbase)

    @pl.when(s >= NS - 8)
    def _():
        for dd in range(EMBED // 8):
            d = (s - (NS - 8)) * (EMBED // 8) + dd
            pltpu.async_copy(
                rel_t_hbm.at[d], srel.at[pl.ds(d * NREL_P, NREL_P)], sem
            ).wait()

    plsc.subcore_barrier()

    for k in range(NSUB):
        for cc in range(SUB // LANES):
            acc[k, pl.ds(cc * LANES, LANES)] = jnp.zeros((LANES,), jnp.float32)

    def d_body(d, _):
        for k in range(NSUB):
            for cc in range(SUB // LANES):
                sl = pl.ds(cc * LANES, LANES)
                rsi[k, sl] = ri[k, sl] + (dbase + d) * NREL_P
        copies = [
            pltpu.async_copy(srel.at[rsi.at[k]], relv.at[k], sem)
            for k in range(NSUB)
        ]

        wait_stream()
        plsc.subcore_barrier()

        for k in range(NSUB):
            copies.append(
                pltpu.async_copy(srow.at[hi.at[k]], hv.at[k], sem))
            copies.append(
                pltpu.async_copy(srow.at[ti.at[k]], tv.at[k], sem))
        for cp in copies:
            cp.wait()

        plsc.subcore_barrier()

        @pl.when(d < D_PER_C - 1)
        def _():
            fire_stream(dbase + d + 1)

        for k in range(NSUB):
            for cc in range(SUB // LANES):
                sl = pl.ds(cc * LANES, LANES)
                acc[k, sl] += jnp.abs(hv[k, sl] + relv[k, sl] - tv[k, sl])
        return 0

    lax.fori_loop(0, D_PER_C, d_body, 0)

    for k in range(NSUB):
        off = c * BATCH + tbase + k * SUB
        pltpu.sync_copy(acc.at[k],
                        out_hbm.at[pl.ds(pl.multiple_of(off, SUB), SUB)])


def kernel(h, r, t, entity_embedding, relation_embedding):
    rel_padded_t = jnp.pad(relation_embedding, ((0, NREL_P - NREL), (0, 0))).T
    tail_t = jnp.pad(entity_embedding[ALIGNED:, :].T,
                     ((0, 0), (0, TAIL - (NENT - ALIGNED))))
    parts = _transe_sc(
        h.astype(jnp.int32),
        r.astype(jnp.int32),
        t.astype(jnp.int32),
        entity_embedding.T,
        rel_padded_t,
        tail_t,
    )
    return -(parts[:BATCH] + parts[BATCH:])
